# baseline (speedup 1.0000x reference)
import functools
import math

import jax
import jax.numpy as jnp
from jax import lax
from jax.experimental import pallas as pl
from jax.experimental.pallas import tpu as pltpu
from jax.experimental.pallas import tpu_sc as plsc

_N = 1000000
_B = 16384
_LN_DECAY = math.log(0.95)
_NC, _NS = 2, 16
_NW = _NC * _NS
_CH = 31264
_LAST_LO = (_NW - 1) * _CH
_TAIL = _N - _LAST_LO
_TAIL_OFF = _LAST_LO - (_N - _CH)
_NV = _B // 16
_LI_BITS = 15
_LI_MASK = (1 << _LI_BITS) - 1


def _body(idx_h, t_h, cnt_h, lt_h, outc_h, outlt_h,
          idx_v, t_v, cnt_v, lt_v, pk_v, newc_v, sem_a, sem_b):
    wid = lax.axis_index("s") * _NC + lax.axis_index("c")
    lo = wid * _CH
    hi = jnp.minimum(lo + _CH, _N)
    span = (hi - lo).astype(jnp.uint32)
    phys = pl.multiple_of(jnp.minimum(lo, _N - _CH), 32)
    loff = lo - phys

    cp1 = pltpu.async_copy(idx_h, idx_v, sem_a)
    cp2 = pltpu.async_copy(t_h, t_v, sem_a)
    cp3 = pltpu.async_copy(cnt_h.at[pl.ds(phys, _CH)], cnt_v, sem_b)
    cp4 = pltpu.async_copy(lt_h.at[pl.ds(phys, _CH)], lt_v, sem_b)
    cp1.wait(); cp2.wait()

    @plsc.parallel_loop(0, _NV, unroll=16, carry=jnp.int32(0))
    def _scan(i, off):
        sl = pl.ds(i * 16, 16)
        iv = idx_v[sl]
        mask = (iv - lo).astype(jnp.uint32) < span
        packed = (t_v[sl] << _LI_BITS) + (iv - phys)
        plsc.store_compressed(pk_v.at[pl.ds(off, 16)], packed, mask=mask)
        return off + plsc.all_reduce_population_count(mask)[0]

    m = _scan
    cp3.wait(); cp4.wait()

    lane = lax.iota(jnp.int32, 16)
    nb = (m + 15) // 16

    @plsc.parallel_loop(0, nb, unroll=2)
    def _compute(i):
        base = i * 16
        sl = pl.ds(base, 16)
        pk = pk_v[sl]
        mask = (lane + base) < m
        li = pk & _LI_MASK
        tv = lax.shift_right_logical(pk, _LI_BITS).astype(jnp.float32)
        cnt = plsc.load_gather(cnt_v, [li], mask=mask)
        ltv = plsc.load_gather(lt_v, [li], mask=mask)
        newc_v[sl] = jnp.exp(jnp.maximum(tv - ltv, 0.0) * _LN_DECAY) * cnt + 1.0

    def step_scatter(i, carry):
        base = i * 16
        pk = pk_v[pl.ds(base, 16)]
        nc = newc_v[pl.ds(base, 16)]
        mask = (lane + base) < m
        li = pk & _LI_MASK
        tv = lax.shift_right_logical(pk, _LI_BITS).astype(jnp.float32)
        plsc.store_scatter(cnt_v, [li], nc, mask=mask)
        plsc.store_scatter(lt_v, [li], tv, mask=mask)
        return carry

    lax.fori_loop(0, nb, step_scatter, 0)

    @pl.when(wid < _NW - 1)
    def _():
        w1 = pltpu.async_copy(cnt_v, outc_h.at[pl.ds(lo, _CH)], sem_a)
        w2 = pltpu.async_copy(lt_v, outlt_h.at[pl.ds(lo, _CH)], sem_b)
        w1.wait(); w2.wait()

    @pl.when(wid == _NW - 1)
    def _():
        w1 = pltpu.async_copy(cnt_v.at[pl.ds(_TAIL_OFF, _TAIL)],
                              outc_h.at[pl.ds(_LAST_LO, _TAIL)], sem_a)
        w2 = pltpu.async_copy(lt_v.at[pl.ds(_TAIL_OFF, _TAIL)],
                              outlt_h.at[pl.ds(_LAST_LO, _TAIL)], sem_b)
        w1.wait(); w2.wait()


@functools.lru_cache(maxsize=1)
def _sc_update():
    return functools.partial(
        pl.kernel,
        out_type=(jax.ShapeDtypeStruct((_N,), jnp.float32),
                  jax.ShapeDtypeStruct((_N,), jnp.float32)),
        mesh=plsc.VectorSubcoreMesh(core_axis_name="c", subcore_axis_name="s",
                                    num_cores=_NC, num_subcores=_NS),
        compiler_params=pltpu.CompilerParams(needs_layout_passes=False),
        scratch_types=[
            pltpu.VMEM((_B,), jnp.int32),
            pltpu.VMEM((_B,), jnp.int32),
            pltpu.VMEM((_CH,), jnp.float32),
            pltpu.VMEM((_CH,), jnp.float32),
            pltpu.VMEM((_B + 16,), jnp.int32),
            pltpu.VMEM((_B + 16,), jnp.float32),
            pltpu.SemaphoreType.DMA,
            pltpu.SemaphoreType.DMA,
        ],
    )(_body)


def kernel(idx, t, count, last_time):
    return _sc_update()(idx.astype(jnp.int32), t.astype(jnp.int32),
                        count, last_time)
